# Initial kernel scaffold; baseline (speedup 1.0000x reference)
import jax
import jax.numpy as jnp
from jax.experimental import pallas as pl


def kernel(x, edge_index, batch, W1, b1, W2, b2, W3, b3, Wl, bl):
    raise NotImplementedError("write your pallas kernel here")


# speedup vs baseline: 12.3541x; 12.3541x over previous
import functools

import jax
import jax.numpy as jnp
from jax import lax
from jax.experimental import pallas as pl
from jax.experimental.pallas import tpu as pltpu
from jax.experimental.pallas import tpu_sc as plsc

N_NODES = 10000
N_EDGES = 320000
NUM_GRAPHS = 16

NC = 2
NS = 16
NW = NC * NS
CHUNK = 128
CPW = 80
E_PAD = CPW * NW * CHUNK
N_CHUNK_ROWS = E_PAD // CHUNK
CPW1 = E_PAD // (NS * CHUNK)
ACC_ROWS = 10240
DUMMY = N_NODES
ZPT = ACC_ROWS // NS // CHUNK
OPT = ACC_ROWS // NS


def _sc_msg_split():
  mesh = plsc.VectorSubcoreMesh(core_axis_name="c", subcore_axis_name="s")

  @functools.partial(
      pl.kernel,
      out_type=jax.ShapeDtypeStruct((2 * ACC_ROWS, 64), jnp.float32),
      mesh=mesh,
      compiler_params=pltpu.CompilerParams(use_tc_tiling_on_sc=False),
      scratch_types=[
          pltpu.VMEM((CPW1, CHUNK), jnp.int32),
          pltpu.VMEM((CPW1, CHUNK), jnp.int32),
          pltpu.VMEM((CHUNK, 64), jnp.float32),
          pltpu.VMEM((CHUNK, 64), jnp.float32),
          pltpu.VMEM_SHARED((ACC_ROWS, 64), jnp.float32),
          pltpu.SemaphoreType.DMA,
      ],
  )
  def msg(h_hbm, src_hbm, dst_hbm, z_hbm, out_hbm, srcv, dstv, zv, rows,
          acc, sem):
    c = lax.axis_index("c")
    s = lax.axis_index("s")
    pltpu.sync_copy(z_hbm, zv)
    for j in range(ZPT):
      pltpu.sync_copy(zv, acc.at[pl.ds(s * (ZPT * CHUNK) + j * CHUNK, CHUNK)])
    pltpu.sync_copy(src_hbm.at[pl.ds(c * N_CHUNK_ROWS + s * CPW1, CPW1)], srcv)
    pltpu.sync_copy(dst_hbm.at[pl.ds(s * CPW1, CPW1)], dstv)
    plsc.subcore_barrier()

    def body(k, carry):
      pltpu.async_copy(h_hbm.at[srcv.at[k]], rows, sem).wait()
      pltpu.sync_copy(rows, acc.at[dstv.at[k]], add=True)
      return carry

    lax.fori_loop(0, CPW1, body, 0)
    plsc.subcore_barrier()
    pltpu.sync_copy(acc.at[pl.ds(s * OPT, OPT)],
                    out_hbm.at[pl.ds(c * ACC_ROWS + s * OPT, OPT)])

  return msg


def _sc_msg_call(F):
  mesh = plsc.VectorSubcoreMesh(core_axis_name="c", subcore_axis_name="s")

  @functools.partial(
      pl.kernel,
      out_type=jax.ShapeDtypeStruct((2 * ACC_ROWS, F), jnp.float32),
      mesh=mesh,
      compiler_params=pltpu.CompilerParams(use_tc_tiling_on_sc=False),
      scratch_types=[
          pltpu.VMEM((CPW, CHUNK), jnp.int32),
          pltpu.VMEM((CPW, CHUNK), jnp.int32),
          pltpu.VMEM((CHUNK, F), jnp.float32),
          pltpu.VMEM((CHUNK, F), jnp.float32),
          pltpu.VMEM_SHARED((ACC_ROWS, F), jnp.float32),
          pltpu.SemaphoreType.DMA,
      ],
  )
  def msg(h_hbm, src_hbm, dst_hbm, z_hbm, out_hbm, srcv, dstv, zv, rows,
          acc, sem):
    c = lax.axis_index("c")
    s = lax.axis_index("s")
    w = c * NS + s
    pltpu.sync_copy(z_hbm, zv)
    for j in range(ZPT):
      pltpu.sync_copy(zv, acc.at[pl.ds(s * (ZPT * CHUNK) + j * CHUNK, CHUNK)])
    pltpu.sync_copy(src_hbm.at[pl.ds(w * CPW, CPW)], srcv)
    pltpu.sync_copy(dst_hbm.at[pl.ds(w * CPW, CPW)], dstv)
    plsc.subcore_barrier()

    def body(k, carry):
      pltpu.async_copy(h_hbm.at[srcv.at[k]], rows, sem).wait()
      pltpu.sync_copy(rows, acc.at[dstv.at[k]], add=True)
      return carry

    lax.fori_loop(0, CPW, body, 0)
    plsc.subcore_barrier()
    pltpu.sync_copy(acc.at[pl.ds(s * OPT, OPT)],
                    out_hbm.at[pl.ds(c * ACC_ROWS + s * OPT, OPT)])

  return msg


def _sc_deg_call():
  mesh = plsc.VectorSubcoreMesh(core_axis_name="c", subcore_axis_name="s")

  @functools.partial(
      pl.kernel,
      out_type=jax.ShapeDtypeStruct((2 * ACC_ROWS, 16), jnp.float32),
      mesh=mesh,
      compiler_params=pltpu.CompilerParams(use_tc_tiling_on_sc=False),
      scratch_types=[
          pltpu.VMEM((CPW, CHUNK), jnp.int32),
          pltpu.VMEM((CHUNK, 16), jnp.float32),
          pltpu.VMEM((CHUNK, 16), jnp.float32),
          pltpu.VMEM_SHARED((ACC_ROWS, 16), jnp.float32),
      ],
  )
  def deg(dst_hbm, ones_hbm, z_hbm, out_hbm, dstv, onesv, zv, acc):
    c = lax.axis_index("c")
    s = lax.axis_index("s")
    w = c * NS + s
    pltpu.sync_copy(z_hbm, zv)
    for j in range(ZPT):
      pltpu.sync_copy(zv, acc.at[pl.ds(s * (ZPT * CHUNK) + j * CHUNK, CHUNK)])
    pltpu.sync_copy(ones_hbm, onesv)
    pltpu.sync_copy(dst_hbm.at[pl.ds(w * CPW, CPW)], dstv)
    plsc.subcore_barrier()

    def body(k, carry):
      pltpu.sync_copy(onesv, acc.at[dstv.at[k]], add=True)
      return carry

    lax.fori_loop(0, CPW, body, 0)
    plsc.subcore_barrier()
    pltpu.sync_copy(acc.at[pl.ds(s * OPT, OPT)],
                    out_hbm.at[pl.ds(c * ACC_ROWS + s * OPT, OPT)])

  return deg


def _tc_first(degp, x, W1):
  def body(degp_ref, x_ref, w_ref, dinv_ref, h_ref):
    p0 = degp_ref[pl.ds(0, N_NODES), :]
    p1 = degp_ref[pl.ds(ACC_ROWS, N_NODES), :]
    deg = 1.0 + p0[:, 0:1] + p1[:, 0:1]
    dinv = lax.rsqrt(deg)
    dinv_ref[...] = dinv
    h = jnp.dot(x_ref[...], w_ref[...], preferred_element_type=jnp.float32)
    h = h * dinv
    h_ref[pl.ds(0, N_NODES), :] = h[:, 0:64]
    h_ref[pl.ds(N_NODES, N_NODES), :] = h[:, 64:128]

  return pl.pallas_call(
      body,
      out_shape=(jax.ShapeDtypeStruct((N_NODES, 1), jnp.float32),
                 jax.ShapeDtypeStruct((2 * N_NODES, 64), jnp.float32)),
  )(degp, x, W1)


def _tc_second(msgp, hsplit, dinv, b, W):
  Fin, Fout = W.shape

  def body(p_ref, hs_ref, dinv_ref, b_ref, w_ref, o_ref):
    dv = dinv_ref[...]
    lo = p_ref[pl.ds(0, N_NODES), :] + hs_ref[pl.ds(0, N_NODES), :]
    hi = (p_ref[pl.ds(ACC_ROWS, N_NODES), :]
          + hs_ref[pl.ds(N_NODES, N_NODES), :])
    tot = jnp.concatenate([lo, hi], axis=1)
    u = jnp.maximum(tot * dv + b_ref[...], 0.0)
    o_ref[...] = jnp.dot(u, w_ref[...],
                         preferred_element_type=jnp.float32) * dv

  return pl.pallas_call(
      body,
      out_shape=jax.ShapeDtypeStruct((N_NODES, Fout), jnp.float32),
  )(msgp, hsplit, dinv, b.reshape(1, Fin), W)


def _tc_mid(msgp, hs, dinv, b, W):
  Fin, Fout = W.shape

  def body(p_ref, hs_ref, dinv_ref, b_ref, w_ref, o_ref):
    dv = dinv_ref[...]
    tot = (p_ref[pl.ds(0, N_NODES), :] + p_ref[pl.ds(ACC_ROWS, N_NODES), :]
           + hs_ref[...])
    u = jnp.maximum(tot * dv + b_ref[...], 0.0)
    o_ref[...] = jnp.dot(u, w_ref[...],
                         preferred_element_type=jnp.float32) * dv

  return pl.pallas_call(
      body,
      out_shape=jax.ShapeDtypeStruct((N_NODES, Fout), jnp.float32),
  )(msgp, hs, dinv, b.reshape(1, Fin), W)


def _tc_final(msgp, hs, dinv, b, batch, Wl, bl):
  F = hs.shape[1]

  def body(p_ref, hs_ref, dinv_ref, b_ref, batch_ref, wl_ref, bl_ref, o_ref):
    tot = (p_ref[pl.ds(0, N_NODES), :] + p_ref[pl.ds(ACC_ROWS, N_NODES), :]
           + hs_ref[...])
    h = jnp.maximum(tot * dinv_ref[...] + b_ref[...], 0.0)
    gids = lax.broadcasted_iota(jnp.int32, (N_NODES, NUM_GRAPHS), 1)
    m = (batch_ref[...] == gids).astype(jnp.float32)
    sums = lax.dot_general(m, h, (((0,), (0,)), ((), ())),
                           preferred_element_type=jnp.float32)
    ones = jnp.ones((N_NODES, 1), jnp.float32)
    counts = lax.dot_general(m, ones, (((0,), (0,)), ((), ())),
                             preferred_element_type=jnp.float32)
    pooled = sums / jnp.maximum(counts, 1.0)
    o_ref[...] = jnp.dot(pooled, wl_ref[...],
                         preferred_element_type=jnp.float32) + bl_ref[...]

  return pl.pallas_call(
      body,
      out_shape=jax.ShapeDtypeStruct((NUM_GRAPHS, 1), jnp.float32),
  )(msgp, hs, dinv, b.reshape(1, F), batch.astype(jnp.int32)[:, None],
    Wl, bl.reshape(1, 1))


def kernel(x, edge_index, batch, W1, b1, W2, b2, W3, b3, Wl, bl):
  src = edge_index[0].astype(jnp.int32)
  dst = edge_index[1].astype(jnp.int32)
  pad = E_PAD - N_EDGES
  src_flat = jnp.concatenate([src, jnp.zeros((pad,), jnp.int32)])
  src2 = src_flat.reshape(N_CHUNK_ROWS, CHUNK)
  src_stacked = jnp.concatenate([src2, src2 + N_NODES], axis=0)
  dst2 = jnp.concatenate([dst, jnp.full((pad,), DUMMY, jnp.int32)])
  dst2 = dst2.reshape(N_CHUNK_ROWS, CHUNK)
  ones16 = jnp.ones((CHUNK, 16), jnp.float32)
  z16 = jnp.zeros((CHUNK, 16), jnp.float32)

  degp = _sc_deg_call()(dst2, ones16, z16)
  dinv, h1split = _tc_first(degp, x, W1)

  p1 = _sc_msg_split()(h1split, src_stacked, dst2,
                       jnp.zeros((CHUNK, 64), jnp.float32))
  h2s = _tc_second(p1, h1split, dinv, b1, W2)

  p2 = _sc_msg_call(64)(h2s, src2, dst2, jnp.zeros((CHUNK, 64), jnp.float32))
  h3s = _tc_mid(p2, h2s, dinv, b2, W3)

  p3 = _sc_msg_call(32)(h3s, src2, dst2, jnp.zeros((CHUNK, 32), jnp.float32))
  return _tc_final(p3, h3s, dinv, b3, batch, Wl, bl)


# speedup vs baseline: 13.1220x; 1.0622x over previous
import functools

import jax
import jax.numpy as jnp
from jax import lax
from jax.experimental import pallas as pl
from jax.experimental.pallas import tpu as pltpu
from jax.experimental.pallas import tpu_sc as plsc

N_NODES = 10000
N_EDGES = 320000
NUM_GRAPHS = 16

NC = 2
NS = 16
NW = NC * NS
CHUNK = 128
CPW = 80
E_PAD = CPW * NW * CHUNK
N_CHUNK_ROWS = E_PAD // CHUNK
CPW1 = E_PAD // (NS * CHUNK)
ACC_ROWS = 10240
DUMMY = N_NODES
ZPT = ACC_ROWS // NS // CHUNK
OPT = ACC_ROWS // NS


def _pipelined_edge_loop(n_chunks, h_hbm, srcv, dstv, rows0, rows1, acc,
                         g0, g1, s0, s1):
  pltpu.async_copy(h_hbm.at[srcv.at[0]], rows0, g0)

  def step(k, first):
    kn = jnp.minimum(k + 1, n_chunks - 1)
    kn2 = jnp.minimum(k + 2, n_chunks - 1)
    pltpu.make_async_copy(h_hbm.at[srcv.at[k]], rows0, g0).wait()
    if not first:
      pltpu.make_async_copy(rows1, acc.at[dstv.at[k - 1]], s1).wait()
    pltpu.async_copy(h_hbm.at[srcv.at[kn]], rows1, g1)
    pltpu.async_copy(rows0, acc.at[dstv.at[k]], s0, add=True)
    pltpu.make_async_copy(h_hbm.at[srcv.at[kn]], rows1, g1).wait()
    pltpu.make_async_copy(rows0, acc.at[dstv.at[k]], s0).wait()
    pltpu.async_copy(h_hbm.at[srcv.at[kn2]], rows0, g0)
    pltpu.async_copy(rows1, acc.at[dstv.at[kn]], s1, add=True)

  step(0, True)

  def body(m, carry):
    step(m * 2, False)
    return carry

  lax.fori_loop(1, n_chunks // 2, body, 0)
  pltpu.make_async_copy(h_hbm.at[srcv.at[n_chunks - 1]], rows0, g0).wait()
  pltpu.make_async_copy(rows1, acc.at[dstv.at[n_chunks - 1]], s1).wait()


def _sc_msg_split():
  mesh = plsc.VectorSubcoreMesh(core_axis_name="c", subcore_axis_name="s")

  @functools.partial(
      pl.kernel,
      out_type=jax.ShapeDtypeStruct((2 * ACC_ROWS, 64), jnp.float32),
      mesh=mesh,
      compiler_params=pltpu.CompilerParams(use_tc_tiling_on_sc=False),
      scratch_types=[
          pltpu.VMEM((CPW1, CHUNK), jnp.int32),
          pltpu.VMEM((CPW1, CHUNK), jnp.int32),
          pltpu.VMEM((CHUNK, 64), jnp.float32),
          pltpu.VMEM((CHUNK, 64), jnp.float32),
          pltpu.VMEM((CHUNK, 64), jnp.float32),
          pltpu.VMEM_SHARED((ACC_ROWS, 64), jnp.float32),
          pltpu.SemaphoreType.DMA,
          pltpu.SemaphoreType.DMA,
          pltpu.SemaphoreType.DMA,
          pltpu.SemaphoreType.DMA,
      ],
  )
  def msg(h_hbm, src_hbm, dst_hbm, z_hbm, out_hbm, srcv, dstv, zv, rows0,
          rows1, acc, g0, g1, s0, s1):
    c = lax.axis_index("c")
    s = lax.axis_index("s")
    pltpu.sync_copy(z_hbm, zv)
    for j in range(ZPT):
      pltpu.sync_copy(zv, acc.at[pl.ds(s * (ZPT * CHUNK) + j * CHUNK, CHUNK)])
    pltpu.sync_copy(src_hbm.at[pl.ds(c * N_CHUNK_ROWS + s * CPW1, CPW1)], srcv)
    pltpu.sync_copy(dst_hbm.at[pl.ds(s * CPW1, CPW1)], dstv)
    plsc.subcore_barrier()
    _pipelined_edge_loop(CPW1, h_hbm, srcv, dstv, rows0, rows1, acc, g0, g1, s0, s1)
    plsc.subcore_barrier()
    pltpu.sync_copy(acc.at[pl.ds(s * OPT, OPT)],
                    out_hbm.at[pl.ds(c * ACC_ROWS + s * OPT, OPT)])

  return msg


def _sc_msg_call(F):
  mesh = plsc.VectorSubcoreMesh(core_axis_name="c", subcore_axis_name="s")

  @functools.partial(
      pl.kernel,
      out_type=jax.ShapeDtypeStruct((2 * ACC_ROWS, F), jnp.float32),
      mesh=mesh,
      compiler_params=pltpu.CompilerParams(use_tc_tiling_on_sc=False),
      scratch_types=[
          pltpu.VMEM((CPW, CHUNK), jnp.int32),
          pltpu.VMEM((CPW, CHUNK), jnp.int32),
          pltpu.VMEM((CHUNK, F), jnp.float32),
          pltpu.VMEM((CHUNK, F), jnp.float32),
          pltpu.VMEM((CHUNK, F), jnp.float32),
          pltpu.VMEM_SHARED((ACC_ROWS, F), jnp.float32),
          pltpu.SemaphoreType.DMA,
          pltpu.SemaphoreType.DMA,
          pltpu.SemaphoreType.DMA,
          pltpu.SemaphoreType.DMA,
      ],
  )
  def msg(h_hbm, src_hbm, dst_hbm, z_hbm, out_hbm, srcv, dstv, zv, rows0,
          rows1, acc, g0, g1, s0, s1):
    c = lax.axis_index("c")
    s = lax.axis_index("s")
    w = c * NS + s
    pltpu.sync_copy(z_hbm, zv)
    for j in range(ZPT):
      pltpu.sync_copy(zv, acc.at[pl.ds(s * (ZPT * CHUNK) + j * CHUNK, CHUNK)])
    pltpu.sync_copy(src_hbm.at[pl.ds(w * CPW, CPW)], srcv)
    pltpu.sync_copy(dst_hbm.at[pl.ds(w * CPW, CPW)], dstv)
    plsc.subcore_barrier()
    _pipelined_edge_loop(CPW, h_hbm, srcv, dstv, rows0, rows1, acc, g0, g1, s0, s1)
    plsc.subcore_barrier()
    pltpu.sync_copy(acc.at[pl.ds(s * OPT, OPT)],
                    out_hbm.at[pl.ds(c * ACC_ROWS + s * OPT, OPT)])

  return msg


def _sc_deg_call():
  mesh = plsc.VectorSubcoreMesh(core_axis_name="c", subcore_axis_name="s")

  @functools.partial(
      pl.kernel,
      out_type=jax.ShapeDtypeStruct((2 * ACC_ROWS, 16), jnp.float32),
      mesh=mesh,
      compiler_params=pltpu.CompilerParams(use_tc_tiling_on_sc=False),
      scratch_types=[
          pltpu.VMEM((CPW, CHUNK), jnp.int32),
          pltpu.VMEM((CHUNK, 16), jnp.float32),
          pltpu.VMEM((CHUNK, 16), jnp.float32),
          pltpu.VMEM_SHARED((ACC_ROWS, 16), jnp.float32),
      ],
  )
  def deg(dst_hbm, ones_hbm, z_hbm, out_hbm, dstv, onesv, zv, acc):
    c = lax.axis_index("c")
    s = lax.axis_index("s")
    w = c * NS + s
    pltpu.sync_copy(z_hbm, zv)
    for j in range(ZPT):
      pltpu.sync_copy(zv, acc.at[pl.ds(s * (ZPT * CHUNK) + j * CHUNK, CHUNK)])
    pltpu.sync_copy(ones_hbm, onesv)
    pltpu.sync_copy(dst_hbm.at[pl.ds(w * CPW, CPW)], dstv)
    plsc.subcore_barrier()

    def body(k, carry):
      pltpu.sync_copy(onesv, acc.at[dstv.at[k]], add=True)
      return carry

    lax.fori_loop(0, CPW, body, 0)
    plsc.subcore_barrier()
    pltpu.sync_copy(acc.at[pl.ds(s * OPT, OPT)],
                    out_hbm.at[pl.ds(c * ACC_ROWS + s * OPT, OPT)])

  return deg


def _tc_first(degp, x, W1):
  def body(degp_ref, x_ref, w_ref, dinv_ref, h_ref):
    p0 = degp_ref[pl.ds(0, N_NODES), :]
    p1 = degp_ref[pl.ds(ACC_ROWS, N_NODES), :]
    deg = 1.0 + p0[:, 0:1] + p1[:, 0:1]
    dinv = 1.0 / jnp.sqrt(deg)
    dinv_ref[...] = dinv
    h = jnp.dot(x_ref[...], w_ref[...], preferred_element_type=jnp.float32)
    h = h * dinv
    h_ref[pl.ds(0, N_NODES), :] = h[:, 0:64]
    h_ref[pl.ds(N_NODES, N_NODES), :] = h[:, 64:128]

  return pl.pallas_call(
      body,
      out_shape=(jax.ShapeDtypeStruct((N_NODES, 1), jnp.float32),
                 jax.ShapeDtypeStruct((2 * N_NODES, 64), jnp.float32)),
  )(degp, x, W1)


def _tc_second(msgp, hsplit, dinv, b, W):
  Fin, Fout = W.shape

  def body(p_ref, hs_ref, dinv_ref, b_ref, w_ref, o_ref):
    dv = dinv_ref[...]
    lo = p_ref[pl.ds(0, N_NODES), :] + hs_ref[pl.ds(0, N_NODES), :]
    hi = (p_ref[pl.ds(ACC_ROWS, N_NODES), :]
          + hs_ref[pl.ds(N_NODES, N_NODES), :])
    tot = jnp.concatenate([lo, hi], axis=1)
    u = jnp.maximum(tot * dv + b_ref[...], 0.0)
    o_ref[...] = jnp.dot(u, w_ref[...],
                         preferred_element_type=jnp.float32) * dv

  return pl.pallas_call(
      body,
      out_shape=jax.ShapeDtypeStruct((N_NODES, Fout), jnp.float32),
  )(msgp, hsplit, dinv, b.reshape(1, Fin), W)


def _tc_mid(msgp, hs, dinv, b, W):
  Fin, Fout = W.shape

  def body(p_ref, hs_ref, dinv_ref, b_ref, w_ref, o_ref):
    dv = dinv_ref[...]
    tot = (p_ref[pl.ds(0, N_NODES), :] + p_ref[pl.ds(ACC_ROWS, N_NODES), :]
           + hs_ref[...])
    u = jnp.maximum(tot * dv + b_ref[...], 0.0)
    o_ref[...] = jnp.dot(u, w_ref[...],
                         preferred_element_type=jnp.float32) * dv

  return pl.pallas_call(
      body,
      out_shape=jax.ShapeDtypeStruct((N_NODES, Fout), jnp.float32),
  )(msgp, hs, dinv, b.reshape(1, Fin), W)


def _tc_final(msgp, hs, dinv, b, batch, Wl, bl):
  F = hs.shape[1]

  def body(p_ref, hs_ref, dinv_ref, b_ref, batch_ref, wl_ref, bl_ref, o_ref):
    tot = (p_ref[pl.ds(0, N_NODES), :] + p_ref[pl.ds(ACC_ROWS, N_NODES), :]
           + hs_ref[...])
    h = jnp.maximum(tot * dinv_ref[...] + b_ref[...], 0.0)
    batch_col = batch_ref[...]
    rows = []
    for g in range(NUM_GRAPHS):
      mask = batch_col == g
      s = jnp.sum(jnp.where(mask, h, 0.0), axis=0, keepdims=True)
      cnt = jnp.sum(jnp.where(mask, 1.0, 0.0), axis=0, keepdims=True)[:, 0:1]
      rows.append(s / jnp.maximum(cnt, 1.0))
    pooled = jnp.concatenate(rows, axis=0)
    o_ref[...] = jnp.dot(pooled, wl_ref[...],
                         preferred_element_type=jnp.float32) + bl_ref[...]

  return pl.pallas_call(
      body,
      out_shape=jax.ShapeDtypeStruct((NUM_GRAPHS, 1), jnp.float32),
  )(msgp, hs, dinv, b.reshape(1, F), batch.astype(jnp.int32)[:, None],
    Wl, bl.reshape(1, 1))


def kernel(x, edge_index, batch, W1, b1, W2, b2, W3, b3, Wl, bl):
  src = edge_index[0].astype(jnp.int32)
  dst = edge_index[1].astype(jnp.int32)
  pad = E_PAD - N_EDGES
  src_flat = jnp.concatenate([src, jnp.zeros((pad,), jnp.int32)])
  src2 = src_flat.reshape(N_CHUNK_ROWS, CHUNK)
  src_stacked = jnp.concatenate([src2, src2 + N_NODES], axis=0)
  dst2 = jnp.concatenate([dst, jnp.full((pad,), DUMMY, jnp.int32)])
  dst2 = dst2.reshape(N_CHUNK_ROWS, CHUNK)
  ones16 = jnp.ones((CHUNK, 16), jnp.float32)
  z16 = jnp.zeros((CHUNK, 16), jnp.float32)

  degp = _sc_deg_call()(dst2, ones16, z16)
  dinv, h1split = _tc_first(degp, x, W1)

  p1 = _sc_msg_split()(h1split, src_stacked, dst2,
                       jnp.zeros((CHUNK, 64), jnp.float32))
  h2s = _tc_second(p1, h1split, dinv, b1, W2)

  p2 = _sc_msg_call(64)(h2s, src2, dst2, jnp.zeros((CHUNK, 64), jnp.float32))
  h3s = _tc_mid(p2, h2s, dinv, b2, W3)

  p3 = _sc_msg_call(32)(h3s, src2, dst2, jnp.zeros((CHUNK, 32), jnp.float32))
  return _tc_final(p3, h3s, dinv, b3, batch, Wl, bl)
